# baseline (device time: 20820 ns/iter reference)
import jax
import jax.numpy as jnp
from jax import lax
from jax.experimental import pallas as pl
from jax.experimental.pallas import tpu as pltpu

CH = 128


def kernel(x):
    m, n = x.shape
    q = m // 4
    nch = q // CH

    def body(x_ref, out_ref, sbuf, acc,
             xs, xr, ys, yr, zs, zr, dsem):
        my_x = lax.axis_index("x")
        my_y = lax.axis_index("y")
        my_z = lax.axis_index("z")
        py = lax.rem(my_y, 2)
        pz = lax.rem(my_z, 2)
        c = 2 * py + pz
        c_y = 2 * (1 - py) + pz
        c_z = 2 * py + (1 - pz)
        c_d = 2 * (1 - py) + (1 - pz)
        partner = (1 - my_x, my_y, my_z)
        ybud = (my_x, my_y + 1 - 2 * py, my_z)
        zbud = (my_x, my_y, my_z + 1 - 2 * pz)

        barrier = pltpu.get_barrier_semaphore()
        for nbr in (partner, ybud, zbud):
            pl.semaphore_signal(
                barrier, inc=1, device_id=nbr,
                device_id_type=pl.DeviceIdType.MESH,
            )
        pl.semaphore_wait(barrier, 3)

        def send(off, ssem, rsem, dev):
            r = pltpu.make_async_remote_copy(
                src_ref=acc.at[pl.ds(off, CH), :],
                dst_ref=acc.at[pl.ds(off, CH), :],
                send_sem=ssem, recv_sem=rsem,
                device_id=dev, device_id_type=pl.DeviceIdType.MESH,
            )
            r.start()
            return r

        def recv_wait(off, rsem):
            r = pltpu.make_async_remote_copy(
                src_ref=acc.at[pl.ds(off, CH), :],
                dst_ref=acc.at[pl.ds(off, CH), :],
                send_sem=dsem, recv_sem=rsem,
                device_id=partner, device_id_type=pl.DeviceIdType.MESH,
            )
            r.wait_recv()

        def add(off):
            out_ref[pl.ds(off, CH), :] = (
                x_ref[pl.ds(off, CH), :].astype(jnp.bfloat16)
                + acc[pl.ds(off, CH), :]
            )

        sbuf[...] = x_ref[pl.ds(c * q, q), :].astype(jnp.bfloat16)
        x_rdmas = []
        for k in range(nch):
            r = pltpu.make_async_remote_copy(
                src_ref=sbuf.at[pl.ds(k * CH, CH), :],
                dst_ref=acc.at[pl.ds(c * q + k * CH, CH), :],
                send_sem=xs.at[k], recv_sem=xr.at[k],
                device_id=partner, device_id_type=pl.DeviceIdType.MESH,
            )
            r.start()
            x_rdmas.append(r)

        fwd = []
        for k in range(nch):
            x_rdmas[k].wait_recv()
            fwd.append(send(c * q + k * CH, ys.at[k], yr.at[k], ybud))
            fwd.append(send(c * q + k * CH, zs.at[k], zr.at[k], zbud))
            add(c * q + k * CH)

        for j in range(2):
            recv_wait(c_z * q + j * CH, zr.at[j])
            fwd.append(send(c_z * q + j * CH, ys.at[4 + j], yr.at[4 + j], ybud))
            add(c_z * q + j * CH)

        for k in range(2):
            recv_wait(c_y * q + k * CH, yr.at[k])
            add(c_y * q + k * CH)

        for k in range(2, 4):
            recv_wait(c_y * q + k * CH, yr.at[k])
            fwd.append(send(c_y * q + k * CH, zs.at[2 + k], zr.at[2 + k], zbud))
            add(c_y * q + k * CH)

        for k in range(2, 4):
            recv_wait(c_z * q + k * CH, zr.at[k])
            add(c_z * q + k * CH)

        for k in range(2):
            recv_wait(c_d * q + k * CH, yr.at[4 + k])
            add(c_d * q + k * CH)
        for k in range(2, 4):
            recv_wait(c_d * q + k * CH, zr.at[2 + k])
            add(c_d * q + k * CH)

        for r in x_rdmas:
            r.wait_send()
        for r in fwd:
            r.wait_send()

    return pl.pallas_call(
        body,
        out_shape=jax.ShapeDtypeStruct((m, n), jnp.bfloat16),
        in_specs=[pl.BlockSpec(memory_space=pltpu.VMEM)],
        out_specs=pl.BlockSpec(memory_space=pltpu.VMEM),
        scratch_shapes=[
            pltpu.VMEM((q, n), jnp.bfloat16),
            pltpu.VMEM((m, n), jnp.bfloat16),
            pltpu.SemaphoreType.DMA((4,)),
            pltpu.SemaphoreType.DMA((4,)),
            pltpu.SemaphoreType.DMA((6,)),
            pltpu.SemaphoreType.DMA((6,)),
            pltpu.SemaphoreType.DMA((6,)),
            pltpu.SemaphoreType.DMA((6,)),
            pltpu.SemaphoreType.DMA,
        ],
        compiler_params=pltpu.CompilerParams(collective_id=0),
    )(x)


# device time: 20665 ns/iter; 1.0075x vs baseline; 1.0075x over previous
import jax
import jax.numpy as jnp
from jax import lax
from jax.experimental import pallas as pl
from jax.experimental.pallas import tpu as pltpu

QCH = (64, 64, 128, 128, 128)
QOFF = (0, 64, 128, 256, 384)
KD = 160


def kernel(x):
    m, n = x.shape
    q = m // 4

    def body(x_ref, out_ref, sbuf, acc,
             xs, xr, ys, yr, zs, zr, dsem):
        my_x = lax.axis_index("x")
        my_y = lax.axis_index("y")
        my_z = lax.axis_index("z")
        py = lax.rem(my_y, 2)
        pz = lax.rem(my_z, 2)
        c = 2 * py + pz
        c_y = 2 * (1 - py) + pz
        c_z = 2 * py + (1 - pz)
        c_d = 2 * (1 - py) + (1 - pz)
        partner = (1 - my_x, my_y, my_z)
        ybud = (my_x, my_y + 1 - 2 * py, my_z)
        zbud = (my_x, my_y, my_z + 1 - 2 * pz)

        barrier = pltpu.get_barrier_semaphore()
        for nbr in (partner, ybud, zbud):
            pl.semaphore_signal(
                barrier, inc=1, device_id=nbr,
                device_id_type=pl.DeviceIdType.MESH,
            )
        pl.semaphore_wait(barrier, 3)

        def send(off, rows, ssem, rsem, dev):
            r = pltpu.make_async_remote_copy(
                src_ref=acc.at[pl.ds(off, rows), :],
                dst_ref=acc.at[pl.ds(off, rows), :],
                send_sem=ssem, recv_sem=rsem,
                device_id=dev, device_id_type=pl.DeviceIdType.MESH,
            )
            r.start()
            return r

        def recv_wait(off, rows, rsem):
            r = pltpu.make_async_remote_copy(
                src_ref=acc.at[pl.ds(off, rows), :],
                dst_ref=acc.at[pl.ds(off, rows), :],
                send_sem=dsem, recv_sem=rsem,
                device_id=partner, device_id_type=pl.DeviceIdType.MESH,
            )
            r.wait_recv()

        def add(off, rows):
            out_ref[pl.ds(off, rows), :] = (
                x_ref[pl.ds(off, rows), :].astype(jnp.bfloat16)
                + acc[pl.ds(off, rows), :]
            )

        sbuf[pl.ds(0, q), :] = x_ref[pl.ds(c * q, q), :].astype(jnp.bfloat16)
        sbuf[pl.ds(q, KD), :] = x_ref[pl.ds(c_d * q, KD), :].astype(jnp.bfloat16)

        x_rdmas = []
        for k in range(5):
            r = pltpu.make_async_remote_copy(
                src_ref=sbuf.at[pl.ds(QOFF[k], QCH[k]), :],
                dst_ref=acc.at[pl.ds(c * q + QOFF[k], QCH[k]), :],
                send_sem=xs.at[k], recv_sem=xr.at[k],
                device_id=partner, device_id_type=pl.DeviceIdType.MESH,
            )
            r.start()
            x_rdmas.append(r)
        r = pltpu.make_async_remote_copy(
            src_ref=sbuf.at[pl.ds(q, KD), :],
            dst_ref=acc.at[pl.ds(c_d * q, KD), :],
            send_sem=xs.at[5], recv_sem=xr.at[5],
            device_id=partner, device_id_type=pl.DeviceIdType.MESH,
        )
        r.start()
        x_rdmas.append(r)

        fwd = []
        for k in range(5):
            x_rdmas[k].wait_recv()
            off = c * q + QOFF[k]
            fwd.append(send(off, QCH[k], ys.at[k], yr.at[k], ybud))
            fwd.append(send(off, QCH[k], zs.at[k], zr.at[k], zbud))
            add(off, QCH[k])
        x_rdmas[5].wait_recv()
        add(c_d * q, KD)

        for k in range(2):
            recv_wait(c_z * q + QOFF[k], QCH[k], zr.at[k])
            add(c_z * q + QOFF[k], QCH[k])
        recv_wait(c_z * q + QOFF[2], QCH[2], zr.at[2])
        fwd.append(send(c_z * q + 160, 96, ys.at[5], yr.at[5], ybud))
        add(c_z * q + QOFF[2], QCH[2])
        recv_wait(c_z * q + QOFF[3], QCH[3], zr.at[3])
        fwd.append(send(c_z * q + 256, 80, ys.at[6], yr.at[6], ybud))
        add(c_z * q + QOFF[3], QCH[3])

        for k in range(3):
            recv_wait(c_y * q + QOFF[k], QCH[k], yr.at[k])
            add(c_y * q + QOFF[k], QCH[k])
        recv_wait(c_y * q + QOFF[3], QCH[3], yr.at[3])
        fwd.append(send(c_y * q + 336, 48, zs.at[5], zr.at[5], zbud))
        add(c_y * q + QOFF[3], QCH[3])
        recv_wait(c_y * q + QOFF[4], QCH[4], yr.at[4])
        fwd.append(send(c_y * q + 384, 128, zs.at[6], zr.at[6], zbud))
        add(c_y * q + QOFF[4], QCH[4])

        recv_wait(c_z * q + QOFF[4], QCH[4], zr.at[4])
        add(c_z * q + QOFF[4], QCH[4])

        recv_wait(c_d * q + 160, 96, yr.at[5])
        add(c_d * q + 160, 96)
        recv_wait(c_d * q + 256, 80, yr.at[6])
        add(c_d * q + 256, 80)
        recv_wait(c_d * q + 336, 48, zr.at[5])
        add(c_d * q + 336, 48)
        recv_wait(c_d * q + 384, 128, zr.at[6])
        add(c_d * q + 384, 128)

        for r in x_rdmas:
            r.wait_send()
        for r in fwd:
            r.wait_send()

    return pl.pallas_call(
        body,
        out_shape=jax.ShapeDtypeStruct((m, n), jnp.bfloat16),
        in_specs=[pl.BlockSpec(memory_space=pltpu.VMEM)],
        out_specs=pl.BlockSpec(memory_space=pltpu.VMEM),
        scratch_shapes=[
            pltpu.VMEM((q + KD, n), jnp.bfloat16),
            pltpu.VMEM((m, n), jnp.bfloat16),
            pltpu.SemaphoreType.DMA((6,)),
            pltpu.SemaphoreType.DMA((6,)),
            pltpu.SemaphoreType.DMA((7,)),
            pltpu.SemaphoreType.DMA((7,)),
            pltpu.SemaphoreType.DMA((7,)),
            pltpu.SemaphoreType.DMA((7,)),
            pltpu.SemaphoreType.DMA,
        ],
        compiler_params=pltpu.CompilerParams(collective_id=0),
    )(x)
